# baseline (speedup 1.0000x reference)
import functools

import jax
import jax.numpy as jnp
from jax import lax
from jax.experimental import pallas as pl
from jax.experimental.pallas import tpu as pltpu
from jax.experimental.pallas import tpu_sc as plsc

_REG = 0.01
_EMB = 32
_NW = 32
_IDX = 128
_BK = 16384

def _dense_body(pt_ref, qt_ref, rt_ref, w_ref,
                pwp_ref, pwq_ref, pwr_ref, spp_ref, spq_ref, spr_ref):
    ones = jnp.ones((1, _EMB), jnp.float32)
    for t, (t_ref, pw_ref, sp_ref) in enumerate(
            ((pt_ref, pwp_ref, spp_ref), (qt_ref, pwq_ref, spq_ref),
             (rt_ref, pwr_ref, spr_ref))):
        x = t_ref[...]
        wt = w_ref[0:1, t * _EMB:(t + 1) * _EMB]
        pw = jax.lax.dot_general(
            wt, x, (((1,), (0,)), ((), ())),
            preferred_element_type=jnp.float32)
        sp = jax.lax.dot_general(
            ones, x * x, (((1,), (0,)), ((), ())),
            preferred_element_type=jnp.float32)
        pw_ref[...] = pw.reshape(pw_ref.shape)
        sp_ref[...] = sp.reshape(sp_ref.shape)


def _combine(B):
    BPW = B // _NW
    G = BPW // _IDX
    mesh = plsc.VectorSubcoreMesh(core_axis_name="c", subcore_axis_name="s")

    @functools.partial(
        pl.kernel,
        mesh=mesh,
        out_type=[
            jax.ShapeDtypeStruct((B,), jnp.float32),
            jax.ShapeDtypeStruct((_NW, 48), jnp.float32),
        ],
        scratch_types=[
            pltpu.VMEM((3, G, _IDX), jnp.int32),
            pltpu.VMEM((BPW,), jnp.float32),
            pltpu.VMEM((BPW,), jnp.float32),
            pltpu.VMEM((BPW,), jnp.float32),
            pltpu.VMEM((48,), jnp.float32),
            pltpu.SemaphoreType.DMA,
        ],
    )
    def k(gidx_h, pwp_h, pwq_h, pwr_h, spp_h, spq_h, spr_h,
          inf_h, parts_h, gidx, pwv, spv, infv, sqv, sem):
        wid = lax.axis_index("c") * 16 + lax.axis_index("s")
        base = wid * BPW
        pltpu.sync_copy(gidx_h.at[wid], gidx)

        for t, (pw_h, sp_h) in enumerate(((pwp_h, spp_h), (pwq_h, spq_h),
                                          (pwr_h, spr_h))):
            cps = []
            for g in range(G):
                cps.append(pltpu.async_copy(
                    pw_h.at[gidx.at[t, g]],
                    pwv.at[pl.ds(g * _IDX, _IDX)], sem))
                cps.append(pltpu.async_copy(
                    sp_h.at[gidx.at[t, g]],
                    spv.at[pl.ds(g * _IDX, _IDX)], sem))
            for c in cps:
                c.wait()

            def chunk(v, sq, t=t):
                sl = pl.ds(v * 16, 16)
                if t == 0:
                    infv[sl] = pwv[sl]
                else:
                    infv[sl] = infv[sl] + pwv[sl]
                return sq + spv[sl]

            sq = lax.fori_loop(0, BPW // 16, chunk,
                               jnp.zeros((16,), jnp.float32))
            sqv[pl.ds(16 * t, 16)] = sq

        pltpu.sync_copy(infv, inf_h.at[pl.ds(base, BPW)])
        pltpu.sync_copy(sqv, parts_h.at[wid])

    return k


def kernel(ps, qs, rs, P, Q, R, W):
    B = ps.shape[0]
    wf = W.reshape(-1).astype(jnp.float32)
    wc = wf / jnp.maximum(jnp.sqrt(jnp.sum(wf * wf)), 1.0)
    nblk = pl.cdiv(P.shape[0], _BK)
    tab_spec = pl.BlockSpec((_EMB, _BK), lambda i: (0, i))
    vec_spec = pl.BlockSpec((_BK,), lambda i: (i,))
    pwp, pwq, pwr, spp, spq, spr = pl.pallas_call(
        _dense_body,
        grid=(nblk,),
        in_specs=[tab_spec, tab_spec, tab_spec,
                  pl.BlockSpec((1, 3 * _EMB), lambda i: (0, 0))],
        out_specs=[vec_spec] * 6,
        out_shape=[jax.ShapeDtypeStruct((P.shape[0],), jnp.float32)] * 6,
    )(P.T, Q.T, R.T, wc.reshape(1, 3 * _EMB))

    idx = jnp.stack([ps, qs, rs]).astype(jnp.int32)
    gidx = idx.reshape(3, _NW, -1, _IDX).transpose(1, 0, 2, 3)
    inf, parts = _combine(B)(gidx, pwp, pwq, pwr, spp, spq, spr)
    s = parts.reshape(_NW, 3, 16).sum(axis=(0, 2))
    regs = _REG * (jnp.sqrt(s[0]) + jnp.sqrt(s[1]) + jnp.sqrt(s[2]))
    return inf.reshape(B, 1), regs


# speedup vs baseline: 1.0725x; 1.0725x over previous
import functools

import jax
import jax.numpy as jnp
from jax import lax
from jax.experimental import pallas as pl
from jax.experimental.pallas import tpu as pltpu
from jax.experimental.pallas import tpu_sc as plsc

_REG = 0.01
_EMB = 32
_NW = 32
_IDX = 128
_BK = 32768

def _dense_body(pt_ref, qt_ref, rt_ref, w_ref,
                pwp_ref, pwq_ref, pwr_ref, spp_ref, spq_ref, spr_ref):
    ones = jnp.ones((1, _EMB), jnp.float32)
    for t, (t_ref, pw_ref, sp_ref) in enumerate(
            ((pt_ref, pwp_ref, spp_ref), (qt_ref, pwq_ref, spq_ref),
             (rt_ref, pwr_ref, spr_ref))):
        x = t_ref[...]
        wt = w_ref[0:1, t * _EMB:(t + 1) * _EMB]
        pw = jax.lax.dot_general(
            wt, x, (((1,), (0,)), ((), ())),
            preferred_element_type=jnp.float32)
        sp = jax.lax.dot_general(
            ones, x * x, (((1,), (0,)), ((), ())),
            preferred_element_type=jnp.float32)
        pw_ref[...] = pw.reshape(pw_ref.shape)
        sp_ref[...] = sp.reshape(sp_ref.shape)


def _combine(B):
    BPW = B // _NW
    G = BPW // _IDX
    mesh = plsc.VectorSubcoreMesh(core_axis_name="c", subcore_axis_name="s")

    @functools.partial(
        pl.kernel,
        mesh=mesh,
        out_type=[
            jax.ShapeDtypeStruct((B,), jnp.float32),
            jax.ShapeDtypeStruct((_NW, 48), jnp.float32),
        ],
        scratch_types=[
            pltpu.VMEM((3 * BPW,), jnp.int32),
            pltpu.VMEM((3 * BPW,), jnp.float32),
            pltpu.VMEM((3 * BPW,), jnp.float32),
            pltpu.VMEM((BPW,), jnp.float32),
            pltpu.VMEM((48,), jnp.float32),
            pltpu.SemaphoreType.DMA,
        ],
    )
    def k(ps_h, qs_h, rs_h, pwp_h, pwq_h, pwr_h, spp_h, spq_h, spr_h,
          inf_h, parts_h, idxv, pwv, spv, infv, sqv, sem):
        wid = lax.axis_index("c") * 16 + lax.axis_index("s")
        base = wid * BPW

        tabs = ((ps_h, pwp_h, spp_h), (qs_h, pwq_h, spq_h),
                (rs_h, pwr_h, spr_h))
        for t, (i_h, _, _) in enumerate(tabs):
            pltpu.sync_copy(i_h.at[pl.ds(base, BPW)],
                            idxv.at[pl.ds(t * BPW, BPW)])
        cps = []
        for t, (_, pw_h, sp_h) in enumerate(tabs):
            for g in range(G):
                sl = pl.ds(t * BPW + g * _IDX, _IDX)
                cps.append(pltpu.async_copy(pw_h.at[idxv.at[sl]],
                                            pwv.at[sl], sem))
                cps.append(pltpu.async_copy(sp_h.at[idxv.at[sl]],
                                            spv.at[sl], sem))
        for c in cps:
            c.wait()
        for t in range(3):

            def chunk(v, sq, t=t):
                sl = pl.ds(t * BPW + v * 16, 16)
                so = pl.ds(v * 16, 16)
                if t == 0:
                    infv[so] = pwv[sl]
                else:
                    infv[so] = infv[so] + pwv[sl]
                return sq + spv[sl]

            sq = lax.fori_loop(0, BPW // 16, chunk,
                               jnp.zeros((16,), jnp.float32))
            sqv[pl.ds(16 * t, 16)] = sq

        pltpu.sync_copy(infv, inf_h.at[pl.ds(base, BPW)])
        pltpu.sync_copy(sqv, parts_h.at[wid])

    return k


def kernel(ps, qs, rs, P, Q, R, W):
    B = ps.shape[0]
    wf = W.reshape(-1).astype(jnp.float32)
    wc = wf / jnp.maximum(jnp.sqrt(jnp.sum(wf * wf)), 1.0)
    nblk = pl.cdiv(P.shape[0], _BK)
    tab_spec = pl.BlockSpec((_EMB, _BK), lambda i: (0, i))
    vec_spec = pl.BlockSpec((_BK,), lambda i: (i,))
    pwp, pwq, pwr, spp, spq, spr = pl.pallas_call(
        _dense_body,
        grid=(nblk,),
        in_specs=[tab_spec, tab_spec, tab_spec,
                  pl.BlockSpec((1, 3 * _EMB), lambda i: (0, 0))],
        out_specs=[vec_spec] * 6,
        out_shape=[jax.ShapeDtypeStruct((P.shape[0],), jnp.float32)] * 6,
    )(P.T, Q.T, R.T, wc.reshape(1, 3 * _EMB))

    inf, parts = _combine(B)(ps.astype(jnp.int32), qs.astype(jnp.int32),
                             rs.astype(jnp.int32),
                             pwp, pwq, pwr, spp, spq, spr)
    s = parts.reshape(_NW, 3, 16).sum(axis=(0, 2))
    regs = _REG * (jnp.sqrt(s[0]) + jnp.sqrt(s[1]) + jnp.sqrt(s[2]))
    return inf.reshape(B, 1), regs


# speedup vs baseline: 1.0998x; 1.0255x over previous
import functools

import jax
import jax.numpy as jnp
from jax import lax
from jax.experimental import pallas as pl
from jax.experimental.pallas import tpu as pltpu
from jax.experimental.pallas import tpu_sc as plsc

_REG = 0.01
_EMB = 32
_NW = 32
_IDX = 128
_BK = 32768

def _dense_body(pt_ref, qt_ref, rt_ref, w_ref,
                pwp_ref, pwq_ref, pwr_ref, spp_ref, spq_ref, spr_ref):
    ones = jnp.ones((1, _EMB), jnp.float32)
    w = w_ref[...]
    wn = w / jnp.maximum(jnp.sqrt(jnp.sum(w * w)), 1.0)
    for t, (t_ref, pw_ref, sp_ref) in enumerate(
            ((pt_ref, pwp_ref, spp_ref), (qt_ref, pwq_ref, spq_ref),
             (rt_ref, pwr_ref, spr_ref))):
        x = t_ref[...]
        wt = wn[0:1, t * _EMB:(t + 1) * _EMB]
        pw = jax.lax.dot_general(
            wt, x, (((1,), (0,)), ((), ())),
            preferred_element_type=jnp.float32)
        sp = jax.lax.dot_general(
            ones, x * x, (((1,), (0,)), ((), ())),
            preferred_element_type=jnp.float32)
        pw_ref[...] = pw.reshape(pw_ref.shape)
        sp_ref[...] = sp.reshape(sp_ref.shape)


def _combine(B):
    BPW = B // _NW
    G = BPW // _IDX
    mesh = plsc.VectorSubcoreMesh(core_axis_name="c", subcore_axis_name="s")

    @functools.partial(
        pl.kernel,
        mesh=mesh,
        out_type=[
            jax.ShapeDtypeStruct((B,), jnp.float32),
            jax.ShapeDtypeStruct((_NW, 48), jnp.float32),
        ],
        scratch_types=[
            pltpu.VMEM((3 * BPW,), jnp.int32),
            pltpu.VMEM((3 * BPW,), jnp.float32),
            pltpu.VMEM((3 * BPW,), jnp.float32),
            pltpu.VMEM((BPW,), jnp.float32),
            pltpu.VMEM((48,), jnp.float32),
            pltpu.SemaphoreType.DMA,
        ],
    )
    def k(ps_h, qs_h, rs_h, pwp_h, pwq_h, pwr_h, spp_h, spq_h, spr_h,
          inf_h, parts_h, idxv, pwv, spv, infv, sqv, sem):
        wid = lax.axis_index("c") * 16 + lax.axis_index("s")
        base = wid * BPW

        tabs = ((ps_h, pwp_h, spp_h), (qs_h, pwq_h, spq_h),
                (rs_h, pwr_h, spr_h))
        for t, (i_h, _, _) in enumerate(tabs):
            pltpu.sync_copy(i_h.at[pl.ds(base, BPW)],
                            idxv.at[pl.ds(t * BPW, BPW)])
        cps = []
        for t, (_, pw_h, sp_h) in enumerate(tabs):
            for g in range(G):
                sl = pl.ds(t * BPW + g * _IDX, _IDX)
                cps.append(pltpu.async_copy(pw_h.at[idxv.at[sl]],
                                            pwv.at[sl], sem))
                cps.append(pltpu.async_copy(sp_h.at[idxv.at[sl]],
                                            spv.at[sl], sem))
        for c in cps:
            c.wait()
        for t in range(3):

            def chunk(v, sq, t=t):
                sl = pl.ds(t * BPW + v * 16, 16)
                so = pl.ds(v * 16, 16)
                if t == 0:
                    infv[so] = pwv[sl]
                else:
                    infv[so] = infv[so] + pwv[sl]
                return sq + spv[sl]

            sq = lax.fori_loop(0, BPW // 16, chunk,
                               jnp.zeros((16,), jnp.float32))
            sqv[pl.ds(16 * t, 16)] = sq

        pltpu.sync_copy(infv, inf_h.at[pl.ds(base, BPW)])
        pltpu.sync_copy(sqv, parts_h.at[wid])

    return k


def kernel(ps, qs, rs, P, Q, R, W):
    B = ps.shape[0]
    nblk = pl.cdiv(P.shape[0], _BK)
    tab_spec = pl.BlockSpec((_EMB, _BK), lambda i: (0, i))
    vec_spec = pl.BlockSpec((_BK,), lambda i: (i,))
    pwp, pwq, pwr, spp, spq, spr = pl.pallas_call(
        _dense_body,
        grid=(nblk,),
        in_specs=[tab_spec, tab_spec, tab_spec,
                  pl.BlockSpec((1, 3 * _EMB), lambda i: (0, 0))],
        out_specs=[vec_spec] * 6,
        out_shape=[jax.ShapeDtypeStruct((P.shape[0],), jnp.float32)] * 6,
    )(P.T, Q.T, R.T, W.astype(jnp.float32))

    inf, parts = _combine(B)(ps.astype(jnp.int32), qs.astype(jnp.int32),
                             rs.astype(jnp.int32),
                             pwp, pwq, pwr, spp, spq, spr)
    s = parts.reshape(_NW, 3, 16).sum(axis=(0, 2))
    regs = _REG * (jnp.sqrt(s[0]) + jnp.sqrt(s[1]) + jnp.sqrt(s[2]))
    return inf.reshape(B, 1), regs


# speedup vs baseline: 1.1031x; 1.0030x over previous
import functools

import jax
import jax.numpy as jnp
from jax import lax
from jax.experimental import pallas as pl
from jax.experimental.pallas import tpu as pltpu
from jax.experimental.pallas import tpu_sc as plsc

_REG = 0.01
_EMB = 32
_NW = 32
_IDX = 128
_BK = 32768

def _dense_body(pt_ref, qt_ref, rt_ref, w_ref,
                pwp_ref, pwq_ref, pwr_ref, spp_ref, spq_ref, spr_ref):
    ones = jnp.ones((1, _EMB), jnp.float32)
    w = w_ref[...]
    wn = w / jnp.maximum(jnp.sqrt(jnp.sum(w * w)), 1.0)
    for t, (t_ref, pw_ref, sp_ref) in enumerate(
            ((pt_ref, pwp_ref, spp_ref), (qt_ref, pwq_ref, spq_ref),
             (rt_ref, pwr_ref, spr_ref))):
        x = t_ref[...]
        wt = wn[0:1, t * _EMB:(t + 1) * _EMB]
        pw = jax.lax.dot_general(
            wt, x, (((1,), (0,)), ((), ())),
            preferred_element_type=jnp.float32)
        sp = jax.lax.dot_general(
            ones, x * x, (((1,), (0,)), ((), ())),
            preferred_element_type=jnp.float32)
        pw_ref[...] = pw.reshape(pw_ref.shape)
        sp_ref[...] = sp.reshape(sp_ref.shape)


def _combine(B):
    BPW = B // _NW
    G = BPW // _IDX
    mesh = plsc.VectorSubcoreMesh(core_axis_name="c", subcore_axis_name="s")

    @functools.partial(
        pl.kernel,
        mesh=mesh,
        out_type=[
            jax.ShapeDtypeStruct((B,), jnp.float32),
            jax.ShapeDtypeStruct((_NW, 48), jnp.float32),
        ],
        scratch_types=[
            pltpu.VMEM((3 * BPW,), jnp.int32),
            pltpu.VMEM((3 * BPW,), jnp.float32),
            pltpu.VMEM((3 * BPW,), jnp.float32),
            pltpu.VMEM((BPW,), jnp.float32),
            pltpu.VMEM((48,), jnp.float32),
            pltpu.SemaphoreType.DMA,
        ],
    )
    def k(ps_h, qs_h, rs_h, pwp_h, pwq_h, pwr_h, spp_h, spq_h, spr_h,
          inf_h, parts_h, idxv, pwv, spv, infv, sqv, sem):
        wid = lax.axis_index("c") * 16 + lax.axis_index("s")
        base = wid * BPW

        tabs = ((ps_h, pwp_h, spp_h), (qs_h, pwq_h, spq_h),
                (rs_h, pwr_h, spr_h))
        for t, (i_h, _, _) in enumerate(tabs):
            pltpu.sync_copy(i_h.at[pl.ds(base, BPW)],
                            idxv.at[pl.ds(t * BPW, BPW)])
        cps = []
        for t, (_, pw_h, sp_h) in enumerate(tabs):
            for g in range(G):
                sl = pl.ds(t * BPW + g * _IDX, _IDX)
                cps.append(pltpu.async_copy(pw_h.at[idxv.at[sl]],
                                            pwv.at[sl], sem))
                cps.append(pltpu.async_copy(sp_h.at[idxv.at[sl]],
                                            spv.at[sl], sem))
        for c in cps:
            c.wait()
        for t in range(3):

            def chunk(v, sq, t=t):
                sl = pl.ds(t * BPW + v * 16, 16)
                so = pl.ds(v * 16, 16)
                if t == 0:
                    infv[so] = pwv[sl]
                else:
                    infv[so] = infv[so] + pwv[sl]
                return sq + spv[sl]

            sq = lax.fori_loop(0, BPW // 16, chunk,
                               jnp.zeros((16,), jnp.float32))
            sqv[pl.ds(16 * t, 16)] = sq

        pltpu.sync_copy(infv, inf_h.at[pl.ds(base, BPW)])
        pltpu.sync_copy(sqv, parts_h.at[wid])

    return k


def kernel(ps, qs, rs, P, Q, R, W):
    B = ps.shape[0]
    nblk = pl.cdiv(P.shape[0], _BK)
    tab_spec = pl.BlockSpec((_EMB, _BK), lambda i: (0, i))
    vec_spec = pl.BlockSpec((_BK,), lambda i: (i,))
    pwp, pwq, pwr, spp, spq, spr = pl.pallas_call(
        _dense_body,
        grid=(nblk,),
        in_specs=[tab_spec, tab_spec, tab_spec,
                  pl.BlockSpec((1, 3 * _EMB), lambda i: (0, 0))],
        out_specs=[vec_spec] * 6,
        out_shape=[jax.ShapeDtypeStruct((P.shape[0],), jnp.float32)] * 6,
        compiler_params=pltpu.CompilerParams(
            dimension_semantics=("parallel",)),
    )(P.T, Q.T, R.T, W.astype(jnp.float32))

    inf, parts = _combine(B)(ps.astype(jnp.int32), qs.astype(jnp.int32),
                             rs.astype(jnp.int32),
                             pwp, pwq, pwr, spp, spq, spr)
    s = parts.reshape(_NW, 3, 16).sum(axis=(0, 2))
    regs = _REG * (jnp.sqrt(s[0]) + jnp.sqrt(s[1]) + jnp.sqrt(s[2]))
    return inf.reshape(B, 1), regs
